# baseline (speedup 1.0000x reference)
import functools
import math

import numpy as np
import jax
import jax.numpy as jnp
from jax import lax
from jax.experimental import pallas as pl
from jax.experimental.pallas import tpu as pltpu
from jax.experimental.pallas import tpu_sc as plsc

_PAD = 0
_SCALE = 512.0 ** 0.5


def _sc_geometry():
    try:
        info = plsc.get_sparse_core_info()
        return info.num_cores, info.num_subcores, info.num_lanes
    except Exception:
        return 2, 16, 16

_NC, _NS, _LANES = _sc_geometry()
_NW = _NC * _NS
_NBUF = 4


def _pos_embedding_np(seq_len, d):
    pos = np.arange(seq_len, dtype=np.float64)[:, None]
    i = np.arange(0, d, 2, dtype=np.float64)
    div = np.exp(-math.log(10000.0) * i / d)
    ang = pos * div[None, :]
    pe = np.zeros((seq_len, d), dtype=np.float32)
    pe[:, 0::2] = np.sin(ang)
    pe[:, 1::2] = np.cos(ang)
    return pe


@functools.lru_cache(maxsize=None)
def _build_gather(B, L, D):
    total = B * L
    rows_w = total // _NW
    nchunk = rows_w // L
    nvec = D // _LANES
    cut = 128

    mesh = plsc.VectorSubcoreMesh(core_axis_name="c", subcore_axis_name="s")

    @functools.partial(
        pl.kernel,
        mesh=mesh,
        out_type=jax.ShapeDtypeStruct((total, D), jnp.float32),
        scratch_types=[
            pltpu.VMEM((L, D), jnp.float32),
        ] + [pltpu.VMEM((1, 1, L), jnp.int32)] * _NBUF
          + [pltpu.VMEM((L, D), jnp.float32)] * _NBUF
          + [pltpu.SemaphoreType.DMA] * (5 * _NBUF),
    )
    def gather_kernel(table_hbm, idx_hbm, pe_hbm, out_hbm, pe_v, *rest):
        idxs = rest[:_NBUF]
        bufs = rest[_NBUF:2 * _NBUF]
        isem = rest[2 * _NBUF:3 * _NBUF]
        gsem_a = rest[3 * _NBUF:4 * _NBUF]
        gsem_b = rest[4 * _NBUF:5 * _NBUF]
        ssem_a = rest[5 * _NBUF:6 * _NBUF]
        ssem_b = rest[6 * _NBUF:]

        wid = lax.axis_index("s") * _NC + lax.axis_index("c")
        pltpu.sync_copy(pe_hbm, pe_v)
        row0 = wid * rows_w

        def idx_start(c, p):
            pltpu.async_copy(
                idx_hbm.at[pl.ds(wid, 1)].at[:, pl.ds(c, 1)], idxs[p],
                isem[p])

        def idx_wait(c, p):
            pltpu.make_async_copy(
                idx_hbm.at[pl.ds(wid, 1)].at[:, pl.ds(c, 1)], idxs[p],
                isem[p]).wait()

        def gather_start(c, p):
            row = idxs[p].at[0, 0]
            pltpu.async_copy(
                table_hbm.at[row.at[pl.ds(0, cut)]],
                bufs[p].at[pl.ds(0, cut)], gsem_a[p])
            pltpu.async_copy(
                table_hbm.at[row.at[pl.ds(cut, L - cut)]],
                bufs[p].at[pl.ds(cut, L - cut)], gsem_b[p])

        def gather_wait_a(c, p):
            row = idxs[p].at[0, 0]
            pltpu.make_async_copy(
                table_hbm.at[row.at[pl.ds(0, cut)]],
                bufs[p].at[pl.ds(0, cut)], gsem_a[p]).wait()

        def gather_wait_b(c, p):
            row = idxs[p].at[0, 0]
            pltpu.make_async_copy(
                table_hbm.at[row.at[pl.ds(cut, L - cut)]],
                bufs[p].at[pl.ds(cut, L - cut)], gsem_b[p]).wait()

        def scatter_start_a(c, p):
            pltpu.async_copy(
                bufs[p].at[pl.ds(0, cut)],
                out_hbm.at[pl.ds(row0 + c * L, cut)], ssem_a[p])

        def scatter_start_b(c, p):
            pltpu.async_copy(
                bufs[p].at[pl.ds(cut, L - cut)],
                out_hbm.at[pl.ds(row0 + c * L + cut, L - cut)], ssem_b[p])

        def scatter_wait(c, p):
            pltpu.make_async_copy(
                bufs[p].at[pl.ds(0, cut)],
                out_hbm.at[pl.ds(row0 + c * L, cut)], ssem_a[p]).wait()
            pltpu.make_async_copy(
                bufs[p].at[pl.ds(cut, L - cut)],
                out_hbm.at[pl.ds(row0 + c * L + cut, L - cut)],
                ssem_b[p]).wait()

        def compute_span(p, lo, hi):
            buf = bufs[p]

            @plsc.parallel_loop(lo, hi, step=1, unroll=2)
            def _(i):
                for v in range(nvec):
                    sl = pl.ds(v * _LANES, _LANES)
                    buf[i, sl] = buf[i, sl] * _SCALE + pe_v[i, sl]

        idx_start(0, 0)
        idx_start(1, 1)
        idx_start(2, 2)
        idx_wait(0, 0)
        gather_start(0, 0)
        idx_wait(1, 1)
        gather_start(1, 1)
        for c in range(nchunk):
            p = c % _NBUF
            if c >= 2:
                scatter_wait(c - 2, (c - 2) % _NBUF)
            if c + 3 <= nchunk - 1:
                idx_start(c + 3, (c + 3) % _NBUF)
            if c + 2 <= nchunk - 1:
                q = (c + 2) % _NBUF
                idx_wait(c + 2, q)
                gather_start(c + 2, q)
            gather_wait_a(c, p)
            compute_span(p, 0, cut)
            scatter_start_a(c, p)
            gather_wait_b(c, p)
            compute_span(p, cut, L)
            scatter_start_b(c, p)
        scatter_wait(nchunk - 2, (nchunk - 2) % _NBUF)
        scatter_wait(nchunk - 1, (nchunk - 1) % _NBUF)

    return gather_kernel


def kernel(inputs, table):
    B, L = inputs.shape
    V, D = table.shape
    pe = jnp.asarray(_pos_embedding_np(L, D))
    idx = inputs.astype(jnp.int32).reshape(_NW, B // _NW, L)
    x = _build_gather(B, L, D)(table, idx, pe)
    x = x.reshape(B, L, D)
    mask = inputs == _PAD
    return (x, mask)


# speedup vs baseline: 1.0198x; 1.0198x over previous
import functools
import math

import numpy as np
import jax
import jax.numpy as jnp
from jax import lax
from jax.experimental import pallas as pl
from jax.experimental.pallas import tpu as pltpu
from jax.experimental.pallas import tpu_sc as plsc

_PAD = 0
_SCALE = 512.0 ** 0.5


def _sc_geometry():
    try:
        info = plsc.get_sparse_core_info()
        return info.num_cores, info.num_subcores, info.num_lanes
    except Exception:
        return 2, 16, 16

_NC, _NS, _LANES = _sc_geometry()
_NW = _NC * _NS
_NBUF = 4


def _pos_embedding_np(seq_len, d):
    pos = np.arange(seq_len, dtype=np.float64)[:, None]
    i = np.arange(0, d, 2, dtype=np.float64)
    div = np.exp(-math.log(10000.0) * i / d)
    ang = pos * div[None, :]
    pe = np.zeros((seq_len, d), dtype=np.float32)
    pe[:, 0::2] = np.sin(ang)
    pe[:, 1::2] = np.cos(ang)
    return pe


@functools.lru_cache(maxsize=None)
def _build_gather(B, L, D):
    total = B * L
    rows_w = total // _NW
    nchunk = rows_w // L
    nvec = D // _LANES
    cut = 128

    mesh = plsc.VectorSubcoreMesh(core_axis_name="c", subcore_axis_name="s")

    @functools.partial(
        pl.kernel,
        mesh=mesh,
        out_type=jax.ShapeDtypeStruct((total, D), jnp.float32),
        scratch_types=[
            pltpu.VMEM((L, D), jnp.float32),
        ] + [pltpu.VMEM((1, 1, L), jnp.int32)] * _NBUF
          + [pltpu.VMEM((L, D), jnp.float32)] * _NBUF
          + [pltpu.SemaphoreType.DMA] * (3 * _NBUF),
    )
    def gather_kernel(table_hbm, idx_hbm, pe_hbm, out_hbm, pe_v, *rest):
        idxs = rest[:_NBUF]
        bufs = rest[_NBUF:2 * _NBUF]
        isem = rest[2 * _NBUF:3 * _NBUF]
        gsem = rest[3 * _NBUF:4 * _NBUF]
        ssem = rest[4 * _NBUF:]

        wid = lax.axis_index("s") * _NC + lax.axis_index("c")
        pltpu.sync_copy(pe_hbm, pe_v)
        row0 = wid * rows_w

        def idx_start(c, p):
            pltpu.async_copy(
                idx_hbm.at[pl.ds(wid, 1)].at[:, pl.ds(c, 1)], idxs[p],
                isem[p])

        def idx_wait(c, p):
            pltpu.make_async_copy(
                idx_hbm.at[pl.ds(wid, 1)].at[:, pl.ds(c, 1)], idxs[p],
                isem[p]).wait()

        def gather_start(c, p):
            row = idxs[p].at[0, 0]
            pltpu.async_copy(
                table_hbm.at[row.at[pl.ds(0, cut)]],
                bufs[p].at[pl.ds(0, cut)], gsem[p])
            pltpu.async_copy(
                table_hbm.at[row.at[pl.ds(cut, L - cut)]],
                bufs[p].at[pl.ds(cut, L - cut)], gsem[p])

        def gather_wait(c, p):
            row = idxs[p].at[0, 0]
            pltpu.make_async_copy(
                table_hbm.at[row.at[pl.ds(0, cut)]],
                bufs[p].at[pl.ds(0, cut)], gsem[p]).wait()
            pltpu.make_async_copy(
                table_hbm.at[row.at[pl.ds(cut, L - cut)]],
                bufs[p].at[pl.ds(cut, L - cut)], gsem[p]).wait()

        def scatter_start(c, p):
            pltpu.async_copy(
                bufs[p], out_hbm.at[pl.ds(row0 + c * L, L)], ssem[p])

        def scatter_wait(c, p):
            pltpu.make_async_copy(
                bufs[p], out_hbm.at[pl.ds(row0 + c * L, L)], ssem[p]).wait()

        def compute(p):
            buf = bufs[p]

            @plsc.parallel_loop(0, L, step=1, unroll=2)
            def _(i):
                for v in range(nvec):
                    sl = pl.ds(v * _LANES, _LANES)
                    buf[i, sl] = buf[i, sl] * _SCALE + pe_v[i, sl]

        idx_start(0, 0)
        idx_start(1, 1)
        idx_start(2, 2)
        idx_wait(0, 0)
        gather_start(0, 0)
        idx_wait(1, 1)
        gather_start(1, 1)
        for c in range(nchunk):
            p = c % _NBUF
            if c >= 2:
                scatter_wait(c - 2, (c - 2) % _NBUF)
            if c + 3 <= nchunk - 1:
                idx_start(c + 3, (c + 3) % _NBUF)
            if c + 2 <= nchunk - 1:
                q = (c + 2) % _NBUF
                idx_wait(c + 2, q)
                gather_start(c + 2, q)
            gather_wait(c, p)
            compute(p)
            scatter_start(c, p)
        scatter_wait(nchunk - 2, (nchunk - 2) % _NBUF)
        scatter_wait(nchunk - 1, (nchunk - 1) % _NBUF)

    return gather_kernel


def kernel(inputs, table):
    B, L = inputs.shape
    V, D = table.shape
    pe = jnp.asarray(_pos_embedding_np(L, D))
    idx = inputs.astype(jnp.int32).reshape(_NW, B // _NW, L)
    x = _build_gather(B, L, D)(table, idx, pe)
    x = x.reshape(B, L, D)
    mask = inputs == _PAD
    return (x, mask)
